# baseline (speedup 1.0000x reference)
import math

import numpy as np

import jax
import jax.numpy as jnp
from jax.experimental import pallas as pl
from jax.experimental.pallas import tpu as pltpu

_KLEN = 31
_PAD = 15
_THRESH = 0.05
_EPS = 1e-8


def _gkern_factor(kernlen=_KLEN, nsig=4):
    interval = (2 * nsig + 1.0) / kernlen
    xs = np.linspace(-nsig - interval / 2.0, nsig + interval / 2.0, kernlen + 1)
    cdf = np.array([0.5 * (1.0 + math.erf(v / math.sqrt(2.0))) for v in xs])
    k1 = np.diff(cdf)
    s = np.sqrt(k1)
    return s / s.sum()


def _band_mats(H, W):
    u = _gkern_factor()
    R = np.zeros((H, H), np.float64)
    for i in range(H):
        for i2 in range(max(0, i - _PAD), min(H, i + _PAD + 1)):
            R[i, i2] = u[i2 - i + _PAD]
    B = np.zeros((W, W), np.float64)
    for j in range(W):
        for j2 in range(max(0, j - _PAD), min(W, j + _PAD + 1)):
            B[j2, j] = u[j2 - j + _PAD]
    ree = jnp.asarray(R[0::2, 0::2], jnp.float32)
    reo = jnp.asarray(R[0::2, 1::2], jnp.float32)
    roe = jnp.asarray(R[1::2, 0::2], jnp.float32)
    roo = jnp.asarray(R[1::2, 1::2], jnp.float32)
    return ree, reo, roe, roo, jnp.asarray(B, jnp.float32)


def _fused_kernel(ree_ref, reo_ref, roe_ref, roo_ref, cband_ref,
                  attn_ref, x_ref, pos_ref, neg_ref, mask_scr):
    @pl.when(pl.program_id(1) == 0)
    def _compute_mask():
        af = attn_ref[0]
        half = af.shape[-1] // 2
        t = af[:, :half]
        u = af[:, half:]
        e = (jnp.dot(ree_ref[...], t, preferred_element_type=jnp.float32)
             + jnp.dot(reo_ref[...], u, preferred_element_type=jnp.float32))
        o = (jnp.dot(roe_ref[...], t, preferred_element_type=jnp.float32)
             + jnp.dot(roo_ref[...], u, preferred_element_type=jnp.float32))
        ce = jnp.dot(e, cband_ref[...], preferred_element_type=jnp.float32)
        co = jnp.dot(o, cband_ref[...], preferred_element_type=jnp.float32)
        conv = jnp.concatenate([ce, co], axis=1)
        mn = jnp.min(conv)
        mx = jnp.max(conv)
        soft = (conv - mn) / (mx - mn + _EPS)
        s = jnp.maximum(soft, af)
        m1 = (s > _THRESH).astype(jnp.float32)
        reps = mask_scr.shape[0] // m1.shape[0]
        mt = m1
        while mt.shape[0] < mask_scr.shape[0]:
            mt = jnp.concatenate([mt, mt], axis=0)
        del reps
        mask_scr[...] = mt

    m = mask_scr[...]
    xb = x_ref[0]
    p = xb * m
    pos_ref[0] = p
    neg_ref[0] = xb - p


def kernel(attention, x):
    N, _, H, W = attention.shape
    C = x.shape[1]
    HF, WF = H // 2, 2 * W

    ree, reo, roe, roo, cband = _band_mats(H, W)
    attn_f = attention.astype(jnp.float32).reshape(N, HF, WF)
    CHF = C * HF
    x_f = x.reshape(N, CHF, WF)

    R = min(CHF, 128 * HF)
    grid = (N, pl.cdiv(CHF, R))

    pos_f, neg_f = pl.pallas_call(
        _fused_kernel,
        out_shape=(jax.ShapeDtypeStruct((N, CHF, WF), x.dtype),
                   jax.ShapeDtypeStruct((N, CHF, WF), x.dtype)),
        grid=grid,
        in_specs=[
            pl.BlockSpec((HF, HF), lambda b, ct: (0, 0)),
            pl.BlockSpec((HF, HF), lambda b, ct: (0, 0)),
            pl.BlockSpec((HF, HF), lambda b, ct: (0, 0)),
            pl.BlockSpec((HF, HF), lambda b, ct: (0, 0)),
            pl.BlockSpec((W, W), lambda b, ct: (0, 0)),
            pl.BlockSpec((1, HF, WF), lambda b, ct: (b, 0, 0)),
            pl.BlockSpec((1, R, WF), lambda b, ct: (b, ct, 0)),
        ],
        out_specs=[
            pl.BlockSpec((1, R, WF), lambda b, ct: (b, ct, 0)),
            pl.BlockSpec((1, R, WF), lambda b, ct: (b, ct, 0)),
        ],
        scratch_shapes=[pltpu.VMEM((R, WF), jnp.float32)],
        compiler_params=pltpu.CompilerParams(
            dimension_semantics=("parallel", "arbitrary"),
            vmem_limit_bytes=56 << 20),
        cost_estimate=pl.CostEstimate(
            flops=int(2 * N * H * W * (H + W) + 2 * N * C * H * W),
            transcendentals=0,
            bytes_accessed=int(4 * (3 * N * C * H * W + N * H * W))),
    )(ree, reo, roe, roo, cband, attn_f, x_f)

    return (pos_f.reshape(N, C, H, W), neg_f.reshape(N, C, H, W))
